# Initial kernel scaffold; baseline (speedup 1.0000x reference)
import jax
import jax.numpy as jnp
from jax.experimental import pallas as pl


def kernel(num_edges, x_ik, x_kj, edge_index_ij, edge_index_ik, edge_index_kj, W, b):
    raise NotImplementedError("write your pallas kernel here")


# speedup vs baseline: 7.6770x; 7.6770x over previous
import jax
import jax.numpy as jnp
from jax.experimental import pallas as pl


def _proj_body(x_ik_ref, x_kj_ref, w1t_ref, w2t_ref, bias_ref, ya_ref, yb_ref):
    ya_ref[...] = jnp.dot(x_ik_ref[...], w1t_ref[...],
                          preferred_element_type=jnp.float32)
    yb_ref[...] = jnp.dot(x_kj_ref[...], w2t_ref[...],
                          preferred_element_type=jnp.float32) + bias_ref[...]


def _project(x_ik, x_kj, W, b):
    E, D_IN = x_ik.shape
    D_OUT = W.shape[0]
    w1t = W[:, :D_IN].T
    w2t = W[:, D_IN:].T
    BLK = 512
    assert E % BLK == 0
    grid = (E // BLK,)
    return pl.pallas_call(
        _proj_body,
        grid=grid,
        in_specs=[
            pl.BlockSpec((BLK, D_IN), lambda i: (i, 0)),
            pl.BlockSpec((BLK, D_IN), lambda i: (i, 0)),
            pl.BlockSpec((D_IN, D_OUT), lambda i: (0, 0)),
            pl.BlockSpec((D_IN, D_OUT), lambda i: (0, 0)),
            pl.BlockSpec((1, D_OUT), lambda i: (0, 0)),
        ],
        out_specs=[
            pl.BlockSpec((BLK, D_OUT), lambda i: (i, 0)),
            pl.BlockSpec((BLK, D_OUT), lambda i: (i, 0)),
        ],
        out_shape=[
            jax.ShapeDtypeStruct((E, D_OUT), jnp.float32),
            jax.ShapeDtypeStruct((E, D_OUT), jnp.float32),
        ],
    )(x_ik, x_kj, w1t, w2t, b.reshape(1, D_OUT))


def kernel(num_edges, x_ik, x_kj, edge_index_ij, edge_index_ik, edge_index_kj, W, b):
    y_ik, y_kj = _project(x_ik, x_kj, W, b)
    g = jax.nn.relu(jnp.take(y_ik, edge_index_ik, axis=0)
                    + jnp.take(y_kj, edge_index_kj, axis=0))
    E = x_ik.shape[0]
    return jax.ops.segment_sum(g, edge_index_ij, num_segments=E)
